# Initial kernel scaffold; baseline (speedup 1.0000x reference)
import functools

import jax
import jax.numpy as jnp
from jax import lax
from jax.experimental import pallas as pl
from jax.experimental.pallas import tpu as pltpu
from jax.experimental.pallas import tpu_sc as plsc

N = 10000
NPAD = 10240
NE = 320000
D = 128
NW = 32
EPW = NE // NW
CH = 80
NCH = EPW // CH
RPT = NPAD // 16

_mesh = plsc.VectorSubcoreMesh(core_axis_name="c", subcore_axis_name="s")


@functools.partial(
    pl.kernel,
    mesh=_mesh,
    out_type=jax.ShapeDtypeStruct((NW, NPAD), jnp.float32),
    scratch_types=[
        pltpu.VMEM((EPW // 16, 16), jnp.int32),
        pltpu.VMEM((NPAD,), jnp.float32),
    ],
)
def _deg_kernel(u_hbm, out_hbm, uidx, degbuf):
    c = lax.axis_index("c")
    s = lax.axis_index("s")
    w = c * 16 + s
    pltpu.sync_copy(u_hbm.at[w], uidx)

    zero16 = jnp.zeros((16,), jnp.float32)

    def _zero(i, carry):
        degbuf[pl.ds(i * 16, 16)] = zero16
        return carry

    lax.fori_loop(0, NPAD // 16, _zero, 0)

    ones16 = jnp.ones((16,), jnp.float32)

    def _acc(i, carry):
        idx = uidx[i, :]
        plsc.addupdate_scatter(degbuf, [idx], ones16)
        return carry

    lax.fori_loop(0, EPW // 16, _acc, 0)
    pltpu.sync_copy(degbuf, out_hbm.at[w])


@functools.partial(
    pl.kernel,
    mesh=_mesh,
    out_type=jax.ShapeDtypeStruct((2, NPAD, D), jnp.float32),
    scratch_types=[
        pltpu.VMEM((NCH, CH), jnp.int32),
        pltpu.VMEM((NCH, CH), jnp.int32),
        pltpu.VMEM((CH, D), jnp.float32),
        pltpu.VMEM_SHARED((NPAD, D), jnp.float32),
        pltpu.SemaphoreType.DMA,
    ],
)
def _scat_kernel(hp_hbm, u_hbm, v_hbm, out_hbm, uidx, vidx, rb, acc, sem):
    c = lax.axis_index("c")
    s = lax.axis_index("s")
    w = c * 16 + s
    pltpu.sync_copy(u_hbm.at[w], uidx)
    pltpu.sync_copy(v_hbm.at[w], vidx)

    zero16 = jnp.zeros((16,), jnp.float32)

    def _zrow(r, carry):
        for j in range(D // 16):
            rb[r, pl.ds(j * 16, 16)] = zero16
        return carry

    lax.fori_loop(0, CH, _zrow, 0)

    base = s * RPT
    for k in range(RPT // CH):
        pltpu.sync_copy(rb, acc.at[pl.ds(base + k * CH, CH)])
    plsc.subcore_barrier()

    def _chunk(ci, carry):
        pltpu.async_copy(hp_hbm.at[uidx.at[ci]], rb, sem).wait()
        pltpu.sync_copy(rb, acc.at[vidx.at[ci]], add=True)
        return carry

    lax.fori_loop(0, NCH, _chunk, 0)
    plsc.subcore_barrier()

    for k in range(RPT // CH):
        pltpu.sync_copy(acc.at[pl.ds(base + k * CH, CH)], rb)
        pltpu.sync_copy(rb, out_hbm.at[c, pl.ds(base + k * CH, CH)])


def _dis_body(degp_ref, dis_ref):
    total = jnp.sum(degp_ref[...], axis=0) + 1.0
    dis_ref[...] = lax.rsqrt(total)[:, None]


def _enc_body(h_ref, w_ref, b_ref, dis_ref, hl_ref, hp_ref):
    hl = jnp.dot(h_ref[...], w_ref[...], preferred_element_type=jnp.float32)
    hl = hl + b_ref[...]
    hl_ref[...] = hl
    hp_ref[...] = hl * dis_ref[...]


def _upd_body(s_ref, hl_ref, dis_ref, w_ref, b_ref, o_ref):
    agg = (s_ref[0] + s_ref[1]) * dis_ref[...]
    pre = jnp.dot(agg + hl_ref[...], w_ref[...], preferred_element_type=jnp.float32)
    o_ref[...] = jnp.maximum(pre + b_ref[...], 0.0)


def _ro_body(h_ref, wo_ref, bo_ref, o_ref):
    rows = lax.broadcasted_iota(jnp.int32, (NPAD, 1), 0)
    hm = jnp.where(rows < N, h_ref[...], 0.0)
    g = jnp.sum(hm, axis=0, keepdims=True) * (1.0 / N)
    o_ref[...] = jnp.dot(g, wo_ref[...], preferred_element_type=jnp.float32) + bo_ref[...]


def _dis_call(degp):
    return pl.pallas_call(
        _dis_body,
        out_shape=jax.ShapeDtypeStruct((NPAD, 1), jnp.float32),
    )(degp)


def _enc_call(h, w, b, dis):
    return pl.pallas_call(
        _enc_body,
        out_shape=(
            jax.ShapeDtypeStruct((NPAD, D), jnp.float32),
            jax.ShapeDtypeStruct((NPAD, D), jnp.float32),
        ),
    )(h, w, b, dis)


def _upd_call(sacc, hl, dis, w, b):
    return pl.pallas_call(
        _upd_body,
        out_shape=jax.ShapeDtypeStruct((NPAD, D), jnp.float32),
    )(sacc, hl, dis, w, b)


def _ro_call(h, wo, bo):
    return pl.pallas_call(
        _ro_body,
        out_shape=jax.ShapeDtypeStruct((1, 1), jnp.float32),
    )(h, wo, bo)


def kernel(H, edge_index, E, W_ne0, b_ne0, W_up0, b_up0, W_ne1, b_ne1,
           W_up1, b_up1, W_ne2, b_ne2, W_up2, b_up2, W_out, b_out):
    u = edge_index[0].reshape(NW, NCH, CH)
    v = edge_index[1].reshape(NW, NCH, CH)
    u16 = edge_index[0].reshape(NW, EPW // 16, 16)

    h = jnp.pad(H, ((0, NPAD - N), (0, 0)))

    degp = _deg_kernel(u16)
    dis = _dis_call(degp)

    layers = [
        (W_ne0, b_ne0, W_up0, b_up0),
        (W_ne1, b_ne1, W_up1, b_up1),
        (W_ne2, b_ne2, W_up2, b_up2),
    ]
    for (wne, bne, wup, bup) in layers:
        hl, hp = _enc_call(h, wne, bne.reshape(1, D), dis)
        sacc = _scat_kernel(hp, u, v)
        h = _upd_call(sacc, hl, dis, wup, bup.reshape(1, D))

    return _ro_call(h, W_out, b_out.reshape(1, 1))


# speedup vs baseline: 12.9319x; 12.9319x over previous
import functools

import jax
import jax.numpy as jnp
from jax import lax
from jax.experimental import pallas as pl
from jax.experimental.pallas import tpu as pltpu
from jax.experimental.pallas import tpu_sc as plsc

N = 10000
NPAD = 10240
NE = 320000
D = 128
NW = 32
EPW = NE // NW
CH = 80
NCH = EPW // CH
RPT = NPAD // 16

_mesh = plsc.VectorSubcoreMesh(core_axis_name="c", subcore_axis_name="s")


@functools.partial(
    pl.kernel,
    mesh=_mesh,
    out_type=jax.ShapeDtypeStruct((NW, NPAD), jnp.float32),
    scratch_types=[
        pltpu.VMEM((EPW // 16, 16), jnp.int32),
        pltpu.VMEM((NPAD,), jnp.float32),
    ],
    compiler_params=pltpu.CompilerParams(needs_layout_passes=False),
)
def _deg_kernel(u_hbm, out_hbm, uidx, degbuf):
    c = lax.axis_index("c")
    s = lax.axis_index("s")
    w = c * 16 + s
    pltpu.sync_copy(u_hbm.at[w], uidx)

    zero16 = jnp.zeros((16,), jnp.float32)

    def _zero(i, carry):
        degbuf[pl.ds(i * 16, 16)] = zero16
        return carry

    lax.fori_loop(0, NPAD // 16, _zero, 0)

    ones16 = jnp.ones((16,), jnp.float32)

    def _acc(i, carry):
        idx = uidx[i, :]
        plsc.addupdate_scatter(degbuf, [idx], ones16)
        return carry

    lax.fori_loop(0, EPW // 16, _acc, 0)
    pltpu.sync_copy(degbuf, out_hbm.at[w])


@functools.partial(
    pl.kernel,
    mesh=_mesh,
    out_type=jax.ShapeDtypeStruct((2, NPAD, D), jnp.float32),
    scratch_types=[
        pltpu.VMEM((NCH, CH), jnp.int32),
        pltpu.VMEM((NCH, CH), jnp.int32),
        pltpu.VMEM((CH, D), jnp.float32),
        pltpu.VMEM_SHARED((NPAD, D), jnp.float32),
        pltpu.SemaphoreType.DMA,
    ],
    compiler_params=pltpu.CompilerParams(needs_layout_passes=False),
)
def _scat_kernel(hp_hbm, u_hbm, v_hbm, out_hbm, uidx, vidx, rb, acc, sem):
    c = lax.axis_index("c")
    s = lax.axis_index("s")
    w = c * 16 + s
    pltpu.sync_copy(u_hbm.at[w], uidx)
    pltpu.sync_copy(v_hbm.at[w], vidx)

    zero16 = jnp.zeros((16,), jnp.float32)

    def _zrow(r, carry):
        for j in range(D // 16):
            rb[r, pl.ds(j * 16, 16)] = zero16
        return carry

    lax.fori_loop(0, CH, _zrow, 0)

    base = s * RPT
    for k in range(RPT // CH):
        pltpu.sync_copy(rb, acc.at[pl.ds(base + k * CH, CH)])
    plsc.subcore_barrier()

    def _chunk(ci, carry):
        pltpu.async_copy(hp_hbm.at[uidx.at[ci]], rb, sem).wait()
        pltpu.sync_copy(rb, acc.at[vidx.at[ci]], add=True)
        return carry

    lax.fori_loop(0, NCH, _chunk, 0)
    plsc.subcore_barrier()

    for k in range(RPT // CH):
        pltpu.sync_copy(acc.at[pl.ds(base + k * CH, CH)], rb)
        pltpu.sync_copy(rb, out_hbm.at[c, pl.ds(base + k * CH, CH)])


def _dis_body(degp_ref, dis_ref):
    total = jnp.sum(degp_ref[...], axis=0) + 1.0
    dis_ref[...] = lax.rsqrt(total)[:, None]


def _enc_body(h_ref, w_ref, b_ref, dis_ref, hl_ref, hp_ref):
    hl = jnp.dot(h_ref[...], w_ref[...], preferred_element_type=jnp.float32, precision=lax.Precision.HIGHEST)
    hl = hl + b_ref[...]
    hl_ref[...] = hl
    hp_ref[...] = hl * dis_ref[...]


def _upd_body(s_ref, hl_ref, dis_ref, w_ref, b_ref, o_ref):
    agg = (s_ref[0] + s_ref[1]) * dis_ref[...]
    pre = jnp.dot(agg + hl_ref[...], w_ref[...], preferred_element_type=jnp.float32, precision=lax.Precision.HIGHEST)
    o_ref[...] = jnp.maximum(pre + b_ref[...], 0.0)


def _ro_body(h_ref, wo_ref, bo_ref, o_ref):
    rows = lax.broadcasted_iota(jnp.int32, (NPAD, 1), 0)
    hm = jnp.where(rows < N, h_ref[...], 0.0)
    g = jnp.sum(hm, axis=0, keepdims=True) * (1.0 / N)
    o_ref[...] = jnp.dot(g, wo_ref[...], preferred_element_type=jnp.float32, precision=lax.Precision.HIGHEST) + bo_ref[...]


def _dis_call(degp):
    return pl.pallas_call(
        _dis_body,
        out_shape=jax.ShapeDtypeStruct((NPAD, 1), jnp.float32),
    )(degp)


def _enc_call(h, w, b, dis):
    return pl.pallas_call(
        _enc_body,
        out_shape=(
            jax.ShapeDtypeStruct((NPAD, D), jnp.float32),
            jax.ShapeDtypeStruct((NPAD, D), jnp.float32),
        ),
    )(h, w, b, dis)


def _upd_call(sacc, hl, dis, w, b):
    return pl.pallas_call(
        _upd_body,
        out_shape=jax.ShapeDtypeStruct((NPAD, D), jnp.float32),
    )(sacc, hl, dis, w, b)


def _ro_call(h, wo, bo):
    return pl.pallas_call(
        _ro_body,
        out_shape=jax.ShapeDtypeStruct((1, 1), jnp.float32),
    )(h, wo, bo)


def kernel(H, edge_index, E, W_ne0, b_ne0, W_up0, b_up0, W_ne1, b_ne1,
           W_up1, b_up1, W_ne2, b_ne2, W_up2, b_up2, W_out, b_out):
    u = edge_index[0].reshape(NW, NCH, CH)
    v = edge_index[1].reshape(NW, NCH, CH)
    u16 = edge_index[0].reshape(NW, EPW // 16, 16)

    h = jnp.pad(H, ((0, NPAD - N), (0, 0)))

    degp = _deg_kernel(u16)
    dis = _dis_call(degp)

    layers = [
        (W_ne0, b_ne0, W_up0, b_up0),
        (W_ne1, b_ne1, W_up1, b_up1),
        (W_ne2, b_ne2, W_up2, b_up2),
    ]
    for (wne, bne, wup, bup) in layers:
        hl, hp = _enc_call(h, wne, bne.reshape(1, D), dis)
        sacc = _scat_kernel(hp, u, v)
        h = _upd_call(sacc, hl, dis, wup, bup.reshape(1, D))

    return _ro_call(h, W_out, b_out.reshape(1, 1))


# speedup vs baseline: 21.1320x; 1.6341x over previous
import functools

import jax
import jax.numpy as jnp
from jax import lax
from jax.experimental import pallas as pl
from jax.experimental.pallas import tpu as pltpu
from jax.experimental.pallas import tpu_sc as plsc

N = 10000
NPAD = 10240
NE = 320000
D = 128
NW = 32
EPW = NE // NW
ECH = 125
CH = 128
NCH = NE // ECH // NW
GCH = NE // ECH
XP = 80
RPT = NPAD // 16

_mesh = plsc.VectorSubcoreMesh(core_axis_name="c", subcore_axis_name="s")


@functools.partial(
    pl.kernel,
    mesh=_mesh,
    out_type=jax.ShapeDtypeStruct((NW, NPAD), jnp.float32),
    scratch_types=[
        pltpu.VMEM((EPW // 16, 16), jnp.int32),
        pltpu.VMEM((NPAD,), jnp.float32),
    ],
    compiler_params=pltpu.CompilerParams(needs_layout_passes=False),
)
def _deg_kernel(u_hbm, out_hbm, uidx, degbuf):
    c = lax.axis_index("c")
    s = lax.axis_index("s")
    w = c * 16 + s
    pltpu.sync_copy(u_hbm.at[w], uidx)

    zero16 = jnp.zeros((16,), jnp.float32)

    def _zero(i, carry):
        degbuf[pl.ds(i * 16, 16)] = zero16
        return carry

    lax.fori_loop(0, NPAD // 16, _zero, 0)

    ones16 = jnp.ones((16,), jnp.float32)

    def _acc(i, carry):
        idx = uidx[i, :]
        plsc.addupdate_scatter(degbuf, [idx], ones16)
        return carry

    lax.fori_loop(0, EPW // 16, _acc, 0)
    pltpu.sync_copy(degbuf, out_hbm.at[w])


@functools.partial(
    pl.kernel,
    mesh=_mesh,
    out_type=jax.ShapeDtypeStruct((2, NPAD, D), jnp.float32),
    scratch_types=[
        [pltpu.VMEM((2, CH), jnp.int32)] * 8,
        [pltpu.VMEM((CH, D), jnp.float32)] * 2,
        pltpu.VMEM_SHARED((NPAD, D), jnp.float32),
        [pltpu.SemaphoreType.DMA] * 8,
        [pltpu.SemaphoreType.DMA] * 2,
        [pltpu.SemaphoreType.DMA] * 2,
    ],
    compiler_params=pltpu.CompilerParams(needs_layout_passes=False),
)
def _scat_kernel(hp_hbm, uv_hbm, out_hbm, uvx, rbs, acc, isem, gsem, ssem):
    c = lax.axis_index("c")
    s = lax.axis_index("s")
    g0 = (c * 16 + s) * NCH

    zero16 = jnp.zeros((16,), jnp.float32)

    def _zrow(r, carry):
        for j in range(D // 16):
            rbs[0][r, pl.ds(j * 16, 16)] = zero16
        return carry

    lax.fori_loop(0, XP, _zrow, 0)

    base = s * RPT
    zsrc = rbs[0].at[pl.ds(0, XP)]
    for k in range(RPT // XP):
        pltpu.async_copy(zsrc, acc.at[pl.ds(base + k * XP, XP)], gsem[0])
    for k in range(RPT // XP):
        pltpu.make_async_copy(zsrc, acc.at[pl.ds(base + k * XP, XP)], gsem[0]).wait()
    plsc.subcore_barrier()

    for j in range(6):
        pltpu.async_copy(uv_hbm.at[g0 + j], uvx[j], isem[j])
    pltpu.make_async_copy(uv_hbm.at[g0], uvx[0], isem[0]).wait()
    pltpu.async_copy(hp_hbm.at[uvx[0].at[0]], rbs[0], gsem[0])

    def _oct(k, carry):
        for j in range(8):
            ci = 8 * k + j
            b = j % 2
            nb = (j + 1) % 2
            i1 = (j + 1) % 8
            i6 = (j + 6) % 8

            @pl.when(ci + 6 < NCH)
            def _fill_idx():
                pltpu.async_copy(uv_hbm.at[g0 + ci + 6], uvx[i6], isem[i6])

            @pl.when(ci + 1 < NCH)
            def _start_next_gather():
                @pl.when(ci >= 1)
                def _wait_prev_scatter():
                    pltpu.make_async_copy(
                        rbs[nb], acc.at[uvx[i1].at[1]], ssem[nb]).wait()
                pltpu.make_async_copy(uv_hbm.at[g0 + ci + 1], uvx[i1], isem[i1]).wait()
                pltpu.async_copy(hp_hbm.at[uvx[i1].at[0]], rbs[nb], gsem[nb])

            pltpu.make_async_copy(hp_hbm.at[uvx[j].at[0]], rbs[b], gsem[b]).wait()
            pltpu.async_copy(rbs[b], acc.at[uvx[j].at[1]], ssem[b], add=True)
        return carry

    lax.fori_loop(0, NCH // 8, _oct, 0)
    for cl in (NCH - 2, NCH - 1):
        pltpu.make_async_copy(
            rbs[cl % 2], acc.at[uvx[cl % 8].at[1]], ssem[cl % 2]).wait()
    plsc.subcore_barrier()

    for k in range(RPT // XP):
        b = k % 2
        dst = rbs[b].at[pl.ds(0, XP)]
        if k >= 2:
            pltpu.make_async_copy(
                dst, out_hbm.at[c, pl.ds(base + (k - 2) * XP, XP)],
                gsem[b]).wait()
        pltpu.sync_copy(acc.at[pl.ds(base + k * XP, XP)], dst)
        pltpu.async_copy(dst, out_hbm.at[c, pl.ds(base + k * XP, XP)], gsem[b])
    for k in range(RPT // XP - 2, RPT // XP):
        b = k % 2
        pltpu.make_async_copy(
            rbs[b].at[pl.ds(0, XP)],
            out_hbm.at[c, pl.ds(base + k * XP, XP)], gsem[b]).wait()


def _dis_body(degp_ref, dis_ref):
    total = jnp.sum(degp_ref[...], axis=0) + 1.0
    dis_ref[...] = lax.rsqrt(total)[:, None]


def _enc_body(h_ref, w_ref, b_ref, dis_ref, hl_ref, hp_ref):
    hl = jnp.dot(h_ref[...], w_ref[...], preferred_element_type=jnp.float32, precision=lax.Precision.HIGHEST)
    hl = hl + b_ref[...]
    hl_ref[...] = hl
    hp_ref[...] = hl * dis_ref[...]


def _upd_body(s_ref, hl_ref, dis_ref, w_ref, b_ref, o_ref):
    agg = (s_ref[0] + s_ref[1]) * dis_ref[...]
    pre = jnp.dot(agg + hl_ref[...], w_ref[...], preferred_element_type=jnp.float32, precision=lax.Precision.HIGHEST)
    o_ref[...] = jnp.maximum(pre + b_ref[...], 0.0)


def _ro_body(h_ref, wo_ref, bo_ref, o_ref):
    rows = lax.broadcasted_iota(jnp.int32, (NPAD, 1), 0)
    hm = jnp.where(rows < N, h_ref[...], 0.0)
    g = jnp.sum(hm, axis=0, keepdims=True) * (1.0 / N)
    o_ref[...] = jnp.dot(g, wo_ref[...], preferred_element_type=jnp.float32, precision=lax.Precision.HIGHEST) + bo_ref[...]


def _dis_call(degp):
    return pl.pallas_call(
        _dis_body,
        out_shape=jax.ShapeDtypeStruct((NPAD, 1), jnp.float32),
    )(degp)


def _enc_call(h, w, b, dis):
    return pl.pallas_call(
        _enc_body,
        out_shape=(
            jax.ShapeDtypeStruct((NPAD, D), jnp.float32),
            jax.ShapeDtypeStruct((NPAD, D), jnp.float32),
        ),
    )(h, w, b, dis)


def _upd_call(sacc, hl, dis, w, b):
    return pl.pallas_call(
        _upd_body,
        out_shape=jax.ShapeDtypeStruct((NPAD, D), jnp.float32),
    )(sacc, hl, dis, w, b)


def _ro_call(h, wo, bo):
    return pl.pallas_call(
        _ro_body,
        out_shape=jax.ShapeDtypeStruct((1, 1), jnp.float32),
    )(h, wo, bo)


def kernel(H, edge_index, E, W_ne0, b_ne0, W_up0, b_up0, W_ne1, b_ne1,
           W_up1, b_up1, W_ne2, b_ne2, W_up2, b_up2, W_out, b_out):
    uc = edge_index[0].reshape(GCH, ECH)
    vc = edge_index[1].reshape(GCH, ECH)
    trash = N + (jnp.arange(GCH, dtype=jnp.int32) % (NPAD - N))
    pad = jnp.broadcast_to(trash[:, None], (GCH, CH - ECH))
    up = jnp.concatenate([uc, pad], axis=1)
    vp = jnp.concatenate([vc, pad], axis=1)
    uv = jnp.stack([up, vp], axis=1)
    u16 = edge_index[0].reshape(NW, EPW // 16, 16)

    h = jnp.pad(H, ((0, NPAD - N), (0, 0)))

    degp = _deg_kernel(u16)
    dis = _dis_call(degp)

    layers = [
        (W_ne0, b_ne0, W_up0, b_up0),
        (W_ne1, b_ne1, W_up1, b_up1),
        (W_ne2, b_ne2, W_up2, b_up2),
    ]
    for (wne, bne, wup, bup) in layers:
        hl, hp = _enc_call(h, wne, bne.reshape(1, D), dis)
        s = _scat_kernel(hp, uv)
        h = _upd_call(s, hl, dis, wup, bup.reshape(1, D))

    return _ro_call(h, W_out, b_out.reshape(1, 1))


# speedup vs baseline: 21.3736x; 1.0114x over previous
import functools

import jax
import jax.numpy as jnp
from jax import lax
from jax.experimental import pallas as pl
from jax.experimental.pallas import tpu as pltpu
from jax.experimental.pallas import tpu_sc as plsc

N = 10000
NPAD = 10240
NE = 320000
D = 128
NW = 32
EPW = NE // NW
ECH = 125
CH = 128
NCH = NE // ECH // NW
GCH = NE // ECH
XP = 80
RPT = NPAD // 16

_mesh = plsc.VectorSubcoreMesh(core_axis_name="c", subcore_axis_name="s")


@functools.partial(
    pl.kernel,
    mesh=_mesh,
    out_type=jax.ShapeDtypeStruct((NW, NPAD), jnp.float32),
    scratch_types=[
        pltpu.VMEM((EPW // 16, 16), jnp.int32),
        pltpu.VMEM((NPAD,), jnp.float32),
    ],
    compiler_params=pltpu.CompilerParams(needs_layout_passes=False),
)
def _deg_kernel(u_hbm, out_hbm, uidx, degbuf):
    c = lax.axis_index("c")
    s = lax.axis_index("s")
    w = c * 16 + s
    pltpu.sync_copy(u_hbm.at[w], uidx)

    zero16 = jnp.zeros((16,), jnp.float32)

    def _zero(i, carry):
        degbuf[pl.ds(i * 16, 16)] = zero16
        return carry

    lax.fori_loop(0, NPAD // 16, _zero, 0)

    ones16 = jnp.ones((16,), jnp.float32)

    def _acc(i, carry):
        idx = uidx[i, :]
        plsc.addupdate_scatter(degbuf, [idx], ones16)
        return carry

    lax.fori_loop(0, EPW // 16, _acc, 0)
    pltpu.sync_copy(degbuf, out_hbm.at[w])


@functools.partial(
    pl.kernel,
    mesh=_mesh,
    out_type=jax.ShapeDtypeStruct((2, NPAD, D), jnp.float32),
    scratch_types=[
        [pltpu.VMEM((2, CH), jnp.int32)] * 8,
        [pltpu.VMEM((CH, D), jnp.float32)] * 2,
        pltpu.VMEM_SHARED((NPAD, D), jnp.float32),
        [pltpu.SemaphoreType.DMA] * 8,
        [pltpu.SemaphoreType.DMA] * 2,
        [pltpu.SemaphoreType.DMA] * 2,
    ],
    compiler_params=pltpu.CompilerParams(needs_layout_passes=False),
)
def _scat_kernel(hp_hbm, uv_hbm, out_hbm, uvx, rbs, acc, isem, gsem, ssem):
    c = lax.axis_index("c")
    s = lax.axis_index("s")
    g0 = (c * 16 + s) * NCH

    zero16 = jnp.zeros((16,), jnp.float32)

    def _zrow(r, carry):
        for j in range(D // 16):
            rbs[0][r, pl.ds(j * 16, 16)] = zero16
        return carry

    lax.fori_loop(0, XP, _zrow, 0)

    base = s * RPT
    zsrc = rbs[0].at[pl.ds(0, XP)]
    for k in range(RPT // XP):
        pltpu.async_copy(zsrc, acc.at[pl.ds(base + k * XP, XP)], gsem[0])
    for k in range(RPT // XP):
        pltpu.make_async_copy(zsrc, acc.at[pl.ds(base + k * XP, XP)], gsem[0]).wait()
    plsc.subcore_barrier()

    for j in range(6):
        pltpu.async_copy(uv_hbm.at[g0 + j], uvx[j], isem[j])
    pltpu.make_async_copy(uv_hbm.at[g0], uvx[0], isem[0]).wait()
    pltpu.async_copy(hp_hbm.at[uvx[0].at[0]], rbs[0], gsem[0])

    def _oct(k, carry):
        for j in range(8):
            ci = 8 * k + j
            b = j % 2
            nb = (j + 1) % 2
            i1 = (j + 1) % 8
            i6 = (j + 6) % 8

            @pl.when(ci + 6 < NCH)
            def _fill_idx():
                pltpu.async_copy(uv_hbm.at[g0 + ci + 6], uvx[i6], isem[i6])

            @pl.when(ci + 1 < NCH)
            def _start_next_gather():
                @pl.when(ci >= 1)
                def _wait_prev_scatter():
                    pltpu.make_async_copy(
                        rbs[nb], acc.at[uvx[i1].at[1]], ssem[nb]).wait()
                pltpu.make_async_copy(uv_hbm.at[g0 + ci + 1], uvx[i1], isem[i1]).wait()
                pltpu.async_copy(hp_hbm.at[uvx[i1].at[0]], rbs[nb], gsem[nb])

            pltpu.make_async_copy(hp_hbm.at[uvx[j].at[0]], rbs[b], gsem[b]).wait()
            pltpu.async_copy(rbs[b], acc.at[uvx[j].at[1]], ssem[b], add=True)
        return carry

    lax.fori_loop(0, NCH // 8, _oct, 0)
    for cl in (NCH - 2, NCH - 1):
        pltpu.make_async_copy(
            rbs[cl % 2], acc.at[uvx[cl % 8].at[1]], ssem[cl % 2]).wait()
    plsc.subcore_barrier()

    for k in range(RPT // XP):
        b = k % 2
        dst = rbs[b].at[pl.ds(0, XP)]
        if k >= 2:
            pltpu.make_async_copy(
                dst, out_hbm.at[c, pl.ds(base + (k - 2) * XP, XP)],
                gsem[b]).wait()
        pltpu.sync_copy(acc.at[pl.ds(base + k * XP, XP)], dst)
        pltpu.async_copy(dst, out_hbm.at[c, pl.ds(base + k * XP, XP)], gsem[b])
    for k in range(RPT // XP - 2, RPT // XP):
        b = k % 2
        pltpu.make_async_copy(
            rbs[b].at[pl.ds(0, XP)],
            out_hbm.at[c, pl.ds(base + k * XP, XP)], gsem[b]).wait()


def _dis_body(degp_ref, dis_ref):
    total = jnp.sum(degp_ref[...], axis=0) + 1.0
    dis_ref[...] = lax.rsqrt(total)[:, None]


def _enc_body(h_ref, w_ref, b_ref, dis_ref, hl_ref, hp_ref):
    hl = jnp.dot(h_ref[...], w_ref[...], preferred_element_type=jnp.float32, precision=lax.Precision.HIGHEST)
    hl = hl + b_ref[...]
    hl_ref[...] = hl
    hp_ref[...] = hl * dis_ref[...]


def _fuse_body(s_ref, hl_ref, dis_ref, wu_ref, bu_ref, wn_ref, bn_ref,
               hl2_ref, hp2_ref):
    agg = (s_ref[0] + s_ref[1]) * dis_ref[...]
    pre = jnp.dot(agg + hl_ref[...], wu_ref[...], preferred_element_type=jnp.float32, precision=lax.Precision.HIGHEST)
    h = jnp.maximum(pre + bu_ref[...], 0.0)
    hl2 = jnp.dot(h, wn_ref[...], preferred_element_type=jnp.float32, precision=lax.Precision.HIGHEST)
    hl2 = hl2 + bn_ref[...]
    hl2_ref[...] = hl2
    hp2_ref[...] = hl2 * dis_ref[...]


def _updro_body(s_ref, hl_ref, dis_ref, wu_ref, bu_ref, wo_ref, bo_ref, o_ref):
    agg = (s_ref[0] + s_ref[1]) * dis_ref[...]
    pre = jnp.dot(agg + hl_ref[...], wu_ref[...], preferred_element_type=jnp.float32, precision=lax.Precision.HIGHEST)
    h = jnp.maximum(pre + bu_ref[...], 0.0)
    rows = lax.broadcasted_iota(jnp.int32, (NPAD, 1), 0)
    hm = jnp.where(rows < N, h, 0.0)
    g = jnp.sum(hm, axis=0, keepdims=True) * (1.0 / N)
    o_ref[...] = jnp.dot(g, wo_ref[...], preferred_element_type=jnp.float32, precision=lax.Precision.HIGHEST) + bo_ref[...]


def _dis_call(degp):
    return pl.pallas_call(
        _dis_body,
        out_shape=jax.ShapeDtypeStruct((NPAD, 1), jnp.float32),
    )(degp)


def _enc_call(h, w, b, dis):
    return pl.pallas_call(
        _enc_body,
        out_shape=(
            jax.ShapeDtypeStruct((NPAD, D), jnp.float32),
            jax.ShapeDtypeStruct((NPAD, D), jnp.float32),
        ),
    )(h, w, b, dis)


def _fuse_call(sacc, hl, dis, wu, bu, wn, bn):
    return pl.pallas_call(
        _fuse_body,
        out_shape=(
            jax.ShapeDtypeStruct((NPAD, D), jnp.float32),
            jax.ShapeDtypeStruct((NPAD, D), jnp.float32),
        ),
    )(sacc, hl, dis, wu, bu, wn, bn)


def _updro_call(sacc, hl, dis, wu, bu, wo, bo):
    return pl.pallas_call(
        _updro_body,
        out_shape=jax.ShapeDtypeStruct((1, 1), jnp.float32),
    )(sacc, hl, dis, wu, bu, wo, bo)


def kernel(H, edge_index, E, W_ne0, b_ne0, W_up0, b_up0, W_ne1, b_ne1,
           W_up1, b_up1, W_ne2, b_ne2, W_up2, b_up2, W_out, b_out):
    uc = edge_index[0].reshape(GCH, ECH)
    vc = edge_index[1].reshape(GCH, ECH)
    trash = N + (jnp.arange(GCH, dtype=jnp.int32) % (NPAD - N))
    pad = jnp.broadcast_to(trash[:, None], (GCH, CH - ECH))
    up = jnp.concatenate([uc, pad], axis=1)
    vp = jnp.concatenate([vc, pad], axis=1)
    uv = jnp.stack([up, vp], axis=1)
    u16 = edge_index[0].reshape(NW, EPW // 16, 16)

    h = jnp.pad(H, ((0, NPAD - N), (0, 0)))

    degp = _deg_kernel(u16)
    dis = _dis_call(degp)

    hl, hp = _enc_call(h, W_ne0, b_ne0.reshape(1, D), dis)
    s = _scat_kernel(hp, uv)
    hl, hp = _fuse_call(s, hl, dis, W_up0, b_up0.reshape(1, D),
                        W_ne1, b_ne1.reshape(1, D))
    s = _scat_kernel(hp, uv)
    hl, hp = _fuse_call(s, hl, dis, W_up1, b_up1.reshape(1, D),
                        W_ne2, b_ne2.reshape(1, D))
    s = _scat_kernel(hp, uv)
    return _updro_call(s, hl, dis, W_up2, b_up2.reshape(1, D),
                       W_out, b_out.reshape(1, 1))


# speedup vs baseline: 21.5495x; 1.0082x over previous
import functools

import jax
import jax.numpy as jnp
from jax import lax
from jax.experimental import pallas as pl
from jax.experimental.pallas import tpu as pltpu
from jax.experimental.pallas import tpu_sc as plsc

N = 10000
NPAD = 10240
NE = 320000
D = 128
NW = 32
EPW = NE // NW
ECH = 125
CH = 128
NCH = NE // ECH // NW
GCH = NE // ECH
XP = 80
RPT = NPAD // 16

_mesh = plsc.VectorSubcoreMesh(core_axis_name="c", subcore_axis_name="s")


@functools.partial(
    pl.kernel,
    mesh=_mesh,
    out_type=jax.ShapeDtypeStruct((NW, NPAD), jnp.float32),
    scratch_types=[
        pltpu.VMEM((EPW // 16, 16), jnp.int32),
        pltpu.VMEM((NPAD,), jnp.float32),
    ],
    compiler_params=pltpu.CompilerParams(needs_layout_passes=False),
)
def _deg_kernel(u_hbm, out_hbm, uidx, degbuf):
    c = lax.axis_index("c")
    s = lax.axis_index("s")
    w = c * 16 + s
    pltpu.sync_copy(u_hbm.at[w], uidx)

    zero16 = jnp.zeros((16,), jnp.float32)

    def _zero(i, carry):
        degbuf[pl.ds(i * 16, 16)] = zero16
        return carry

    lax.fori_loop(0, NPAD // 16, _zero, 0)

    ones16 = jnp.ones((16,), jnp.float32)

    def _acc(i, carry):
        idx = uidx[i, :]
        plsc.addupdate_scatter(degbuf, [idx], ones16)
        return carry

    lax.fori_loop(0, EPW // 16, _acc, 0)
    pltpu.sync_copy(degbuf, out_hbm.at[w])


@functools.partial(
    pl.kernel,
    mesh=_mesh,
    out_type=jax.ShapeDtypeStruct((2, NPAD, D), jnp.float32),
    scratch_types=[
        [pltpu.VMEM((2, CH), jnp.int32)] * 8,
        [pltpu.VMEM((CH, D), jnp.float32)] * 2,
        pltpu.VMEM((XP, D), jnp.float32),
        pltpu.VMEM_SHARED((NPAD, D), jnp.float32),
        [pltpu.SemaphoreType.DMA] * 8,
        [pltpu.SemaphoreType.DMA] * 2,
        [pltpu.SemaphoreType.DMA] * 2,
        pltpu.SemaphoreType.DMA,
    ],
    compiler_params=pltpu.CompilerParams(needs_layout_passes=False),
)
def _scat_kernel(hp_hbm, uv_hbm, out_hbm, uvx, rbs, zbuf, acc, isem, gsem,
                 ssem, zsem):
    c = lax.axis_index("c")
    s = lax.axis_index("s")
    g0 = (c * 16 + s) * NCH

    zero16 = jnp.zeros((16,), jnp.float32)

    def _zrow(r, carry):
        for j in range(D // 16):
            zbuf[r, pl.ds(j * 16, 16)] = zero16
        return carry

    lax.fori_loop(0, XP, _zrow, 0)

    base = s * RPT
    for k in range(RPT // XP):
        pltpu.async_copy(zbuf, acc.at[pl.ds(base + k * XP, XP)], zsem)

    for j in range(6):
        pltpu.async_copy(uv_hbm.at[g0 + j], uvx[j], isem[j])
    pltpu.make_async_copy(uv_hbm.at[g0], uvx[0], isem[0]).wait()
    pltpu.async_copy(hp_hbm.at[uvx[0].at[0]], rbs[0], gsem[0])

    for k in range(RPT // XP):
        pltpu.make_async_copy(zbuf, acc.at[pl.ds(base + k * XP, XP)], zsem).wait()
    plsc.subcore_barrier()

    def _oct(k, carry):
        for j in range(8):
            ci = 8 * k + j
            b = j % 2
            nb = (j + 1) % 2
            i1 = (j + 1) % 8
            i6 = (j + 6) % 8

            @pl.when(ci + 6 < NCH)
            def _fill_idx():
                pltpu.async_copy(uv_hbm.at[g0 + ci + 6], uvx[i6], isem[i6])

            @pl.when(ci + 1 < NCH)
            def _start_next_gather():
                @pl.when(ci >= 1)
                def _wait_prev_scatter():
                    pltpu.make_async_copy(
                        rbs[nb], acc.at[uvx[i1].at[1]], ssem[nb]).wait()
                pltpu.make_async_copy(uv_hbm.at[g0 + ci + 1], uvx[i1], isem[i1]).wait()
                pltpu.async_copy(hp_hbm.at[uvx[i1].at[0]], rbs[nb], gsem[nb])

            pltpu.make_async_copy(hp_hbm.at[uvx[j].at[0]], rbs[b], gsem[b]).wait()
            pltpu.async_copy(rbs[b], acc.at[uvx[j].at[1]], ssem[b], add=True)
        return carry

    lax.fori_loop(0, NCH // 8, _oct, 0)
    for cl in (NCH - 2, NCH - 1):
        pltpu.make_async_copy(
            rbs[cl % 2], acc.at[uvx[cl % 8].at[1]], ssem[cl % 2]).wait()
    plsc.subcore_barrier()

    for k in range(RPT // XP):
        b = k % 2
        dst = rbs[b].at[pl.ds(0, XP)]
        if k >= 2:
            pltpu.make_async_copy(
                dst, out_hbm.at[c, pl.ds(base + (k - 2) * XP, XP)],
                gsem[b]).wait()
        pltpu.sync_copy(acc.at[pl.ds(base + k * XP, XP)], dst)
        pltpu.async_copy(dst, out_hbm.at[c, pl.ds(base + k * XP, XP)], gsem[b])
    for k in range(RPT // XP - 2, RPT // XP):
        b = k % 2
        pltpu.make_async_copy(
            rbs[b].at[pl.ds(0, XP)],
            out_hbm.at[c, pl.ds(base + k * XP, XP)], gsem[b]).wait()


def _dis_body(degp_ref, dis_ref):
    total = jnp.sum(degp_ref[...], axis=0) + 1.0
    dis_ref[...] = lax.rsqrt(total)[:, None]


def _enc_body(h_ref, w_ref, b_ref, dis_ref, hl_ref, hp_ref):
    hl = jnp.dot(h_ref[...], w_ref[...], preferred_element_type=jnp.float32, precision=lax.Precision.HIGHEST)
    hl = hl + b_ref[...]
    hl_ref[...] = hl
    hp_ref[...] = hl * dis_ref[...]


def _fuse_body(s_ref, hl_ref, dis_ref, wu_ref, bu_ref, wn_ref, bn_ref,
               hl2_ref, hp2_ref):
    agg = (s_ref[0] + s_ref[1]) * dis_ref[...]
    pre = jnp.dot((agg + hl_ref[...]), wu_ref[...], preferred_element_type=jnp.float32, precision=lax.Precision.HIGHEST)
    h = jnp.maximum(pre + bu_ref[...], 0.0)
    hl2 = jnp.dot(h, wn_ref[...], preferred_element_type=jnp.float32, precision=lax.Precision.HIGHEST)
    hl2 = hl2 + bn_ref[...]
    hl2_ref[...] = hl2
    hp2_ref[...] = hl2 * dis_ref[...]


def _updro_body(s_ref, hl_ref, dis_ref, wu_ref, bu_ref, wo_ref, bo_ref, o_ref):
    agg = (s_ref[0] + s_ref[1]) * dis_ref[...]
    pre = jnp.dot((agg + hl_ref[...]), wu_ref[...], preferred_element_type=jnp.float32, precision=lax.Precision.HIGHEST)
    h = jnp.maximum(pre + bu_ref[...], 0.0)
    rows = lax.broadcasted_iota(jnp.int32, (NPAD, 1), 0)
    hm = jnp.where(rows < N, h, 0.0)
    g = jnp.sum(hm, axis=0, keepdims=True) * (1.0 / N)
    o_ref[...] = jnp.dot(g, wo_ref[...], preferred_element_type=jnp.float32, precision=lax.Precision.HIGHEST) + bo_ref[...]


def _dis_call(degp):
    return pl.pallas_call(
        _dis_body,
        out_shape=jax.ShapeDtypeStruct((NPAD, 1), jnp.float32),
    )(degp)


def _enc_call(h, w, b, dis):
    return pl.pallas_call(
        _enc_body,
        out_shape=(
            jax.ShapeDtypeStruct((NPAD, D), jnp.float32),
            jax.ShapeDtypeStruct((NPAD, D), jnp.float32),
        ),
    )(h, w, b, dis)


def _fuse_call(sacc, hl, dis, wu, bu, wn, bn):
    return pl.pallas_call(
        _fuse_body,
        out_shape=(
            jax.ShapeDtypeStruct((NPAD, D), jnp.float32),
            jax.ShapeDtypeStruct((NPAD, D), jnp.float32),
        ),
    )(sacc, hl, dis, wu, bu, wn, bn)


def _updro_call(sacc, hl, dis, wu, bu, wo, bo):
    return pl.pallas_call(
        _updro_body,
        out_shape=jax.ShapeDtypeStruct((1, 1), jnp.float32),
    )(sacc, hl, dis, wu, bu, wo, bo)


def kernel(H, edge_index, E, W_ne0, b_ne0, W_up0, b_up0, W_ne1, b_ne1,
           W_up1, b_up1, W_ne2, b_ne2, W_up2, b_up2, W_out, b_out):
    uc = edge_index[0].reshape(GCH, ECH)
    vc = edge_index[1].reshape(GCH, ECH)
    trash = N + (jnp.arange(GCH, dtype=jnp.int32) % (NPAD - N))
    pad = jnp.broadcast_to(trash[:, None], (GCH, CH - ECH))
    up = jnp.concatenate([uc, pad], axis=1)
    vp = jnp.concatenate([vc, pad], axis=1)
    uv = jnp.stack([up, vp], axis=1)
    u16 = edge_index[0].reshape(NW, EPW // 16, 16)

    h = jnp.pad(H, ((0, NPAD - N), (0, 0)))

    degp = _deg_kernel(u16)
    dis = _dis_call(degp)

    hl, hp = _enc_call(h, W_ne0, b_ne0.reshape(1, D), dis)
    s = _scat_kernel(hp, uv)
    hl, hp = _fuse_call(s, hl, dis, W_up0, b_up0.reshape(1, D),
                        W_ne1, b_ne1.reshape(1, D))
    s = _scat_kernel(hp, uv)
    hl, hp = _fuse_call(s, hl, dis, W_up1, b_up1.reshape(1, D),
                        W_ne2, b_ne2.reshape(1, D))
    s = _scat_kernel(hp, uv)
    return _updro_call(s, hl, dis, W_up2, b_up2.reshape(1, D),
                       W_out, b_out.reshape(1, 1))


# speedup vs baseline: 21.8471x; 1.0138x over previous
import functools

import jax
import jax.numpy as jnp
from jax import lax
from jax.experimental import pallas as pl
from jax.experimental.pallas import tpu as pltpu
from jax.experimental.pallas import tpu_sc as plsc

N = 10000
NPAD = 10240
NE = 320000
D = 128
NW = 32
EPW = NE // NW
ECH = 125
CH = 128
NCH = NE // ECH // NW
GCH = NE // ECH
XP = 80
RPT = NPAD // 16

_mesh = plsc.VectorSubcoreMesh(core_axis_name="c", subcore_axis_name="s")


@functools.partial(
    pl.kernel,
    mesh=_mesh,
    out_type=jax.ShapeDtypeStruct((NW, NPAD), jnp.float32),
    scratch_types=[
        pltpu.VMEM((EPW // 16, 16), jnp.int32),
        pltpu.VMEM((NPAD,), jnp.float32),
    ],
    compiler_params=pltpu.CompilerParams(needs_layout_passes=False),
)
def _deg_kernel(u_hbm, out_hbm, uidx, degbuf):
    c = lax.axis_index("c")
    s = lax.axis_index("s")
    w = c * 16 + s
    pltpu.sync_copy(u_hbm.at[w], uidx)

    zero16 = jnp.zeros((16,), jnp.float32)

    def _zero(i, carry):
        degbuf[pl.ds(i * 16, 16)] = zero16
        return carry

    lax.fori_loop(0, NPAD // 16, _zero, 0)

    ones16 = jnp.ones((16,), jnp.float32)

    def _acc(i, carry):
        idx = uidx[i, :]
        plsc.addupdate_scatter(degbuf, [idx], ones16)
        return carry

    lax.fori_loop(0, EPW // 16, _acc, 0)
    pltpu.sync_copy(degbuf, out_hbm.at[w])


@functools.partial(
    pl.kernel,
    mesh=_mesh,
    out_type=jax.ShapeDtypeStruct((2, NPAD, D), jnp.float32),
    scratch_types=[
        [pltpu.VMEM((2, CH), jnp.int32)] * 8,
        [pltpu.VMEM((CH, D), jnp.float32)] * 2,
        pltpu.VMEM((XP, D), jnp.float32),
        pltpu.VMEM_SHARED((NPAD, D), jnp.float32),
        [pltpu.SemaphoreType.DMA] * 8,
        [pltpu.SemaphoreType.DMA] * 2,
        [pltpu.SemaphoreType.DMA] * 2,
        pltpu.SemaphoreType.DMA,
    ],
    compiler_params=pltpu.CompilerParams(needs_layout_passes=False),
)
def _scat_kernel(hp_hbm, uv_hbm, out_hbm, uvx, rbs, zbuf, acc, isem, gsem,
                 ssem, zsem):
    c = lax.axis_index("c")
    s = lax.axis_index("s")
    g0 = (c * 16 + s) * NCH

    zero16 = jnp.zeros((16,), jnp.float32)

    def _zrow(r, carry):
        for j in range(D // 16):
            zbuf[r, pl.ds(j * 16, 16)] = zero16
        return carry

    lax.fori_loop(0, XP, _zrow, 0)

    base = s * RPT
    for k in range(RPT // XP):
        pltpu.async_copy(zbuf, acc.at[pl.ds(base + k * XP, XP)], zsem)

    for j in range(6):
        pltpu.async_copy(uv_hbm.at[g0 + j], uvx[j], isem[j])
    pltpu.make_async_copy(uv_hbm.at[g0], uvx[0], isem[0]).wait()
    pltpu.async_copy(hp_hbm.at[uvx[0].at[0]], rbs[0], gsem[0])

    for k in range(RPT // XP):
        pltpu.make_async_copy(zbuf, acc.at[pl.ds(base + k * XP, XP)], zsem).wait()
    plsc.subcore_barrier()

    def _oct(k, carry):
        for j in range(8):
            ci = 8 * k + j
            b = j % 2
            nb = (j + 1) % 2
            i1 = (j + 1) % 8
            i6 = (j + 6) % 8

            @pl.when(ci + 6 < NCH)
            def _fill_idx():
                pltpu.async_copy(uv_hbm.at[g0 + ci + 6], uvx[i6], isem[i6])

            @pl.when(ci + 1 < NCH)
            def _start_next_gather():
                @pl.when(ci >= 1)
                def _wait_prev_scatter():
                    pltpu.make_async_copy(
                        rbs[nb], acc.at[uvx[i1].at[1]], ssem[nb]).wait()
                pltpu.make_async_copy(uv_hbm.at[g0 + ci + 1], uvx[i1], isem[i1]).wait()
                pltpu.async_copy(hp_hbm.at[uvx[i1].at[0]], rbs[nb], gsem[nb])

            pltpu.make_async_copy(hp_hbm.at[uvx[j].at[0]], rbs[b], gsem[b]).wait()
            pltpu.async_copy(rbs[b], acc.at[uvx[j].at[1]], ssem[b], add=True)
        return carry

    lax.fori_loop(0, NCH // 8, _oct, 0)
    for cl in (NCH - 2, NCH - 1):
        pltpu.make_async_copy(
            rbs[cl % 2], acc.at[uvx[cl % 8].at[1]], ssem[cl % 2]).wait()
    plsc.subcore_barrier()

    for k in range(RPT // XP):
        b = k % 2
        dst = rbs[b].at[pl.ds(0, XP)]
        if k >= 2:
            pltpu.make_async_copy(
                dst, out_hbm.at[c, pl.ds(base + (k - 2) * XP, XP)],
                gsem[b]).wait()
        pltpu.sync_copy(acc.at[pl.ds(base + k * XP, XP)], dst)
        pltpu.async_copy(dst, out_hbm.at[c, pl.ds(base + k * XP, XP)], gsem[b])
    for k in range(RPT // XP - 2, RPT // XP):
        b = k % 2
        pltpu.make_async_copy(
            rbs[b].at[pl.ds(0, XP)],
            out_hbm.at[c, pl.ds(base + k * XP, XP)], gsem[b]).wait()


def _enc_body(h_ref, w_ref, b_ref, degp_ref, hl_ref, hp_ref, dis_ref):
    dis = lax.rsqrt(jnp.sum(degp_ref[...], axis=0) + 1.0)[:, None]
    dis_ref[...] = dis
    hl = jnp.dot(h_ref[...], w_ref[...], preferred_element_type=jnp.float32, precision=lax.Precision.HIGHEST)
    hl = hl + b_ref[...]
    hl_ref[...] = hl
    hp_ref[...] = hl * dis


def _fuse_body(s_ref, hl_ref, dis_ref, wu_ref, bu_ref, wn_ref, bn_ref,
               hl2_ref, hp2_ref):
    agg = (s_ref[0] + s_ref[1]) * dis_ref[...]
    pre = jnp.dot((agg + hl_ref[...]), wu_ref[...], preferred_element_type=jnp.float32, precision=lax.Precision.HIGHEST)
    h = jnp.maximum(pre + bu_ref[...], 0.0)
    hl2 = jnp.dot(h, wn_ref[...], preferred_element_type=jnp.float32, precision=lax.Precision.HIGHEST)
    hl2 = hl2 + bn_ref[...]
    hl2_ref[...] = hl2
    hp2_ref[...] = hl2 * dis_ref[...]


def _updro_body(s_ref, hl_ref, dis_ref, wu_ref, bu_ref, wo_ref, bo_ref, o_ref):
    agg = (s_ref[0] + s_ref[1]) * dis_ref[...]
    pre = jnp.dot((agg + hl_ref[...]), wu_ref[...], preferred_element_type=jnp.float32, precision=lax.Precision.HIGHEST)
    h = jnp.maximum(pre + bu_ref[...], 0.0)
    rows = lax.broadcasted_iota(jnp.int32, (NPAD, 1), 0)
    hm = jnp.where(rows < N, h, 0.0)
    g = jnp.sum(hm, axis=0, keepdims=True) * (1.0 / N)
    o_ref[...] = jnp.dot(g, wo_ref[...], preferred_element_type=jnp.float32, precision=lax.Precision.HIGHEST) + bo_ref[...]


def _enc_call(h, w, b, degp):
    return pl.pallas_call(
        _enc_body,
        out_shape=(
            jax.ShapeDtypeStruct((NPAD, D), jnp.float32),
            jax.ShapeDtypeStruct((NPAD, D), jnp.float32),
            jax.ShapeDtypeStruct((NPAD, 1), jnp.float32),
        ),
    )(h, w, b, degp)


def _fuse_call(sacc, hl, dis, wu, bu, wn, bn):
    return pl.pallas_call(
        _fuse_body,
        out_shape=(
            jax.ShapeDtypeStruct((NPAD, D), jnp.float32),
            jax.ShapeDtypeStruct((NPAD, D), jnp.float32),
        ),
    )(sacc, hl, dis, wu, bu, wn, bn)


def _updro_call(sacc, hl, dis, wu, bu, wo, bo):
    return pl.pallas_call(
        _updro_body,
        out_shape=jax.ShapeDtypeStruct((1, 1), jnp.float32),
    )(sacc, hl, dis, wu, bu, wo, bo)


def kernel(H, edge_index, E, W_ne0, b_ne0, W_up0, b_up0, W_ne1, b_ne1,
           W_up1, b_up1, W_ne2, b_ne2, W_up2, b_up2, W_out, b_out):
    uc = edge_index[0].reshape(GCH, ECH)
    vc = edge_index[1].reshape(GCH, ECH)
    trash = N + (jnp.arange(GCH, dtype=jnp.int32) % (NPAD - N))
    pad = jnp.broadcast_to(trash[:, None], (GCH, CH - ECH))
    up = jnp.concatenate([uc, pad], axis=1)
    vp = jnp.concatenate([vc, pad], axis=1)
    uv = jnp.stack([up, vp], axis=1)
    u16 = edge_index[0].reshape(NW, EPW // 16, 16)

    h = jnp.pad(H, ((0, NPAD - N), (0, 0)))

    degp = _deg_kernel(u16)
    hl, hp, dis = _enc_call(h, W_ne0, b_ne0.reshape(1, D), degp)
    s = _scat_kernel(hp, uv)
    hl, hp = _fuse_call(s, hl, dis, W_up0, b_up0.reshape(1, D),
                        W_ne1, b_ne1.reshape(1, D))
    s = _scat_kernel(hp, uv)
    hl, hp = _fuse_call(s, hl, dis, W_up1, b_up1.reshape(1, D),
                        W_ne2, b_ne2.reshape(1, D))
    s = _scat_kernel(hp, uv)
    return _updro_call(s, hl, dis, W_up2, b_up2.reshape(1, D),
                       W_out, b_out.reshape(1, 1))
